# baseline (speedup 1.0000x reference)
import jax
import jax.numpy as jnp
from jax.experimental import pallas as pl
from jax.experimental.pallas import tpu as pltpu

BATCH = 4096
DIM = 4096
BLK = 32
GRID = BATCH // BLK
ROW_BYTES = DIM * 4


def _tc_body(uidx_ref, iidx_ref, u_tab, i_tab, u_out, i_out, usem, isem):
    g = pl.program_id(0)

    @pl.when(g > 0)
    def _drain_prev():
        for r in range(BLK):
            pltpu.make_async_copy(u_tab.at[pl.ds(0, 1)],
                                  u_out.at[pl.ds(0, 1)], usem).wait()
            pltpu.make_async_copy(i_tab.at[pl.ds(0, 1)],
                                  i_out.at[pl.ds(0, 1)], isem).wait()

    for r in range(BLK):
        row = g * BLK + r
        pltpu.make_async_copy(u_tab.at[pl.ds(uidx_ref[row], 1)],
                              u_out.at[pl.ds(row, 1)], usem).start()
        pltpu.make_async_copy(i_tab.at[pl.ds(iidx_ref[row], 1)],
                              i_out.at[pl.ds(row, 1)], isem).start()

    @pl.when(g == GRID - 1)
    def _drain_last():
        for r in range(BLK):
            pltpu.make_async_copy(u_tab.at[pl.ds(0, 1)],
                                  u_out.at[pl.ds(0, 1)], usem).wait()
            pltpu.make_async_copy(i_tab.at[pl.ds(0, 1)],
                                  i_out.at[pl.ds(0, 1)], isem).wait()


@jax.jit
def _tc_gather(users, items, user_table, item_table):
    grid_spec = pltpu.PrefetchScalarGridSpec(
        num_scalar_prefetch=2,
        grid=(GRID,),
        in_specs=[
            pl.BlockSpec(memory_space=pl.ANY),
            pl.BlockSpec(memory_space=pl.ANY),
        ],
        out_specs=[
            pl.BlockSpec(memory_space=pl.ANY),
            pl.BlockSpec(memory_space=pl.ANY),
        ],
        scratch_shapes=[pltpu.SemaphoreType.DMA, pltpu.SemaphoreType.DMA],
    )
    return pl.pallas_call(
        _tc_body,
        grid_spec=grid_spec,
        out_shape=[
            jax.ShapeDtypeStruct((BATCH, DIM), jnp.float32),
            jax.ShapeDtypeStruct((BATCH, DIM), jnp.float32),
        ],
    )(users, items, user_table, item_table)


def kernel(users, items, user_table, item_table):
    u_repr, i_repr = _tc_gather(users, items, user_table, item_table)
    return (u_repr, i_repr)


# speedup vs baseline: 34.8876x; 34.8876x over previous
import functools

import jax
import jax.numpy as jnp
from jax import lax
from jax.experimental import pallas as pl
from jax.experimental.pallas import tpu as pltpu
from jax.experimental.pallas import tpu_sc as plsc

BATCH = 4096
DIM = 4096
NUM_CORES = 2
NUM_SUBCORES = 16
NUM_WORKERS = NUM_CORES * NUM_SUBCORES
BPW = BATCH // NUM_WORKERS
CHUNK = 8
NCHUNK = BPW // CHUNK

_MESH = plsc.VectorSubcoreMesh(
    core_axis_name="c", subcore_axis_name="s",
    num_cores=NUM_CORES, num_subcores=NUM_SUBCORES)


@functools.partial(
    pl.kernel,
    out_type=(
        jax.ShapeDtypeStruct((BATCH, DIM), jnp.float32),
        jax.ShapeDtypeStruct((BATCH, DIM), jnp.float32),
    ),
    mesh=_MESH,
    scratch_types=[
        pltpu.VMEM((BPW,), jnp.int32),
        pltpu.VMEM((BPW,), jnp.int32),
        pltpu.VMEM((CHUNK, DIM), jnp.float32),
        pltpu.VMEM((CHUNK, DIM), jnp.float32),
        pltpu.SemaphoreType.DMA,
        pltpu.SemaphoreType.DMA,
        pltpu.SemaphoreType.DMA,
        pltpu.SemaphoreType.DMA,
    ],
)
def _gather2(users_hbm, items_hbm, u_tab, i_tab, u_out, i_out,
             uidx, iidx, ubuf, ibuf, ugsem, igsem, ussem, issem):
    wid = lax.axis_index("s") * NUM_CORES + lax.axis_index("c")
    base = wid * BPW
    pltpu.sync_copy(users_hbm.at[pl.ds(base, BPW)], uidx)
    pltpu.sync_copy(items_hbm.at[pl.ds(base, BPW)], iidx)

    def gstart(tab, idx, buf, sem, c):
        pltpu.async_copy(tab.at[idx.at[pl.ds(c * CHUNK, CHUNK)]], buf, sem)

    def gwait(buf, sem):
        pltpu.make_async_copy(u_tab.at[pl.ds(0, CHUNK)], buf, sem).wait()

    def sstart(out, buf, sem, c):
        pltpu.async_copy(buf, out.at[pl.ds(base + c * CHUNK, CHUNK)], sem)

    def swait(out, buf, sem):
        pltpu.make_async_copy(buf, out.at[pl.ds(base, CHUNK)], sem).wait()

    gstart(u_tab, uidx, ubuf, ugsem, 0)
    gstart(i_tab, iidx, ibuf, igsem, 0)

    @pl.loop(0, NCHUNK - 1)
    def _steady(c):
        gwait(ubuf, ugsem)
        sstart(u_out, ubuf, ussem, c)
        gwait(ibuf, igsem)
        sstart(i_out, ibuf, issem, c)
        swait(u_out, ubuf, ussem)
        gstart(u_tab, uidx, ubuf, ugsem, c + 1)
        swait(i_out, ibuf, issem)
        gstart(i_tab, iidx, ibuf, igsem, c + 1)

    gwait(ubuf, ugsem)
    sstart(u_out, ubuf, ussem, NCHUNK - 1)
    gwait(ibuf, igsem)
    sstart(i_out, ibuf, issem, NCHUNK - 1)
    swait(u_out, ubuf, ussem)
    swait(i_out, ibuf, issem)


def kernel(users, items, user_table, item_table):
    u_repr, i_repr = _gather2(users, items, user_table, item_table)
    return (u_repr, i_repr)
